# Initial kernel scaffold; baseline (speedup 1.0000x reference)
import jax
import jax.numpy as jnp
from jax.experimental import pallas as pl


def kernel(y_pred, y_true):
    raise NotImplementedError("write your pallas kernel here")


# speedup vs baseline: 1.0054x; 1.0054x over previous
import jax
import jax.numpy as jnp
from jax.experimental import pallas as pl
from jax.experimental.pallas import tpu as pltpu

_BLOCK_ROWS = 256


def _loss_body(nblocks, p_ref, t_ref, out_ref, acc_ref):
    i = pl.program_id(0)

    @pl.when(i == 0)
    def _init():
        acc_ref[0, 0] = 0.0
        acc_ref[0, 1] = 0.0

    p = p_ref[...]
    t = t_ref[...]
    mask = t != 0.0
    d = jnp.where(mask, p - t, 0.0)
    acc_ref[0, 0] += jnp.sum(d * d)
    acc_ref[0, 1] += jnp.sum(mask.astype(jnp.float32))

    @pl.when(i == nblocks - 1)
    def _fin():
        cnt = acc_ref[0, 1]
        out_ref[0, 0] = jnp.where(
            cnt > 0.0, acc_ref[0, 0] / jnp.maximum(cnt, 1.0), 0.0
        )


def kernel(y_pred, y_true):
    n, d = y_pred.shape
    nblocks = n // _BLOCK_ROWS
    out = pl.pallas_call(
        lambda p, t, o, a: _loss_body(nblocks, p, t, o, a),
        grid=(nblocks,),
        in_specs=[
            pl.BlockSpec((_BLOCK_ROWS, d), lambda i: (i, 0)),
            pl.BlockSpec((_BLOCK_ROWS, d), lambda i: (i, 0)),
        ],
        out_specs=pl.BlockSpec(memory_space=pltpu.SMEM),
        out_shape=jax.ShapeDtypeStruct((1, 1), jnp.float32),
        scratch_shapes=[pltpu.SMEM((1, 2), jnp.float32)],
    )(y_pred, y_true)
    return out[0, 0]
